# Initial kernel scaffold; baseline (speedup 1.0000x reference)
import jax
import jax.numpy as jnp
from jax.experimental import pallas as pl


def kernel(x, edge_index, W1, b1, W2, b2, Wl, bl):
    raise NotImplementedError("write your pallas kernel here")


# speedup vs baseline: 12.2266x; 12.2266x over previous
import functools

import jax
import jax.numpy as jnp
from jax import lax
from jax.experimental import pallas as pl
from jax.experimental.pallas import tpu as pltpu
from jax.experimental.pallas import tpu_sc as plsc

N = 10000
E = 320000
D = 128

NC = 2
NS = 16
NW = NC * NS
EPW = E // NW
C = 80
NCHUNK = EPW // C
RPT = 624
REM_BASE = NS * RPT
REM = N - REM_BASE

_mesh = plsc.VectorSubcoreMesh(
    core_axis_name="c", subcore_axis_name="s", num_cores=NC, num_subcores=NS
)


def _tile_rows_copy(s, fn):
    fn(s * RPT, RPT)

    @pl.when(s == NS - 1)
    def _():
        fn(REM_BASE, REM)


def _deg_body(dst_hbm, ones_hbm, zeros_hbm, onesC_hbm, out_hbm,
              idx_d, ones_v, acc, sem):
    c = lax.axis_index("c")
    s = lax.axis_index("s")
    wid = s * NC + c

    @pl.when(c == 0)
    def _():
        _tile_rows_copy(s, lambda rb, nr: pltpu.sync_copy(
            ones_hbm.at[pl.ds(rb, nr)], acc.at[pl.ds(rb, nr)]))

    @pl.when(c != 0)
    def _():
        _tile_rows_copy(s, lambda rb, nr: pltpu.sync_copy(
            zeros_hbm.at[pl.ds(rb, nr)], acc.at[pl.ds(rb, nr)]))

    pltpu.sync_copy(onesC_hbm, ones_v)
    plsc.subcore_barrier()
    ebase = wid * EPW

    @pl.loop(0, NCHUNK)
    def _(i):
        base = ebase + i * C
        pltpu.sync_copy(dst_hbm.at[pl.ds(base, C)], idx_d)
        pltpu.sync_copy(ones_v, acc.at[idx_d], add=True)

    plsc.subcore_barrier()
    _tile_rows_copy(s, lambda rb, nr: pltpu.async_copy(
        acc.at[pl.ds(rb, nr)], out_hbm.at[c, pl.ds(rb, nr)], sem).wait())


@jax.jit
def _sc_degrees(dst, ones128, zeros128, onesC):
    kern = pl.kernel(
        _deg_body,
        out_type=jax.ShapeDtypeStruct((NC, N, D), jnp.float32),
        mesh=_mesh,
        scratch_types=[
            pltpu.VMEM((C,), jnp.int32),
            pltpu.VMEM((C, D), jnp.float32),
            pltpu.VMEM_SHARED((N, D), jnp.float32),
            pltpu.SemaphoreType.DMA,
        ],
    )
    return kern(dst, ones128, zeros128, onesC)


def _agg_body(g_hbm, src_hbm, dst_hbm, zeros_hbm, out_hbm,
              idx_s, idx_d, rows, acc, sem):
    c = lax.axis_index("c")
    s = lax.axis_index("s")
    wid = s * NC + c

    @pl.when(c == 0)
    def _():
        _tile_rows_copy(s, lambda rb, nr: pltpu.sync_copy(
            g_hbm.at[pl.ds(rb, nr)], acc.at[pl.ds(rb, nr)]))

    @pl.when(c != 0)
    def _():
        _tile_rows_copy(s, lambda rb, nr: pltpu.sync_copy(
            zeros_hbm.at[pl.ds(rb, nr)], acc.at[pl.ds(rb, nr)]))

    plsc.subcore_barrier()
    ebase = wid * EPW

    @pl.loop(0, NCHUNK)
    def _(i):
        base = ebase + i * C
        pltpu.sync_copy(src_hbm.at[pl.ds(base, C)], idx_s)
        pltpu.sync_copy(dst_hbm.at[pl.ds(base, C)], idx_d)
        pltpu.async_copy(g_hbm.at[idx_s], rows, sem).wait()
        pltpu.sync_copy(rows, acc.at[idx_d], add=True)

    plsc.subcore_barrier()
    _tile_rows_copy(s, lambda rb, nr: pltpu.async_copy(
        acc.at[pl.ds(rb, nr)], out_hbm.at[c, pl.ds(rb, nr)], sem).wait())


@jax.jit
def _sc_aggregate(g, src, dst, zeros128):
    kern = pl.kernel(
        _agg_body,
        out_type=jax.ShapeDtypeStruct((NC, N, D), jnp.float32),
        mesh=_mesh,
        scratch_types=[
            pltpu.VMEM((C,), jnp.int32),
            pltpu.VMEM((C,), jnp.int32),
            pltpu.VMEM((C, D), jnp.float32),
            pltpu.VMEM_SHARED((N, D), jnp.float32),
            pltpu.SemaphoreType.DMA,
        ],
    )
    return kern(g, src, dst, zeros128)


BM = 1000
GRID = N // BM


def _mm_body(x_ref, w_ref, o_ref):
    o_ref[...] = jnp.dot(x_ref[...], w_ref[...],
                         preferred_element_type=jnp.float32)


def _scale_body(u_ref, degp_ref, o_ref):
    deg = degp_ref[0, :, 0:1] + degp_ref[1, :, 0:1]
    o_ref[...] = u_ref[...] * lax.rsqrt(deg)


def _comb2_body(sp_ref, degp_ref, b_ref, w_ref, o_ref):
    deg = degp_ref[0, :, 0:1] + degp_ref[1, :, 0:1]
    dis = lax.rsqrt(deg)
    h = jnp.maximum((sp_ref[0] + sp_ref[1]) * dis + b_ref[...], 0.0)
    o_ref[...] = jnp.dot(h, w_ref[...],
                         preferred_element_type=jnp.float32) * dis


def _out_body(sp_ref, degp_ref, b_ref, w_ref, bl_ref, o_ref):
    deg = degp_ref[0, :, 0:1] + degp_ref[1, :, 0:1]
    dis = lax.rsqrt(deg)
    h = (sp_ref[0] + sp_ref[1]) * dis + b_ref[...]
    o_ref[...] = jnp.dot(h, w_ref[...],
                         preferred_element_type=jnp.float32) + bl_ref[...]


_spec_rows = pl.BlockSpec((BM, D), lambda i: (i, 0))
_spec_part = pl.BlockSpec((NC, BM, D), lambda i: (0, i, 0))
_spec_deg = pl.BlockSpec((NC, BM, D), lambda i: (0, i, 0))
_spec_w = pl.BlockSpec((D, D), lambda i: (0, 0))
_spec_b = pl.BlockSpec((1, D), lambda i: (0, 0))
_f32 = jnp.float32


@jax.jit
def _tc_mm(x, w):
    return pl.pallas_call(
        _mm_body, grid=(GRID,),
        in_specs=[_spec_rows, _spec_w], out_specs=_spec_rows,
        out_shape=jax.ShapeDtypeStruct((N, D), _f32),
    )(x, w)


@jax.jit
def _tc_scale(u, degp):
    return pl.pallas_call(
        _scale_body, grid=(GRID,),
        in_specs=[_spec_rows, _spec_deg], out_specs=_spec_rows,
        out_shape=jax.ShapeDtypeStruct((N, D), _f32),
    )(u, degp)


@jax.jit
def _tc_comb2(sp, degp, b1, w2):
    return pl.pallas_call(
        _comb2_body, grid=(GRID,),
        in_specs=[_spec_part, _spec_deg, _spec_b, _spec_w],
        out_specs=_spec_rows,
        out_shape=jax.ShapeDtypeStruct((N, D), _f32),
    )(sp, degp, b1, w2)


@jax.jit
def _tc_out(sp, degp, b2, wl, bl):
    return pl.pallas_call(
        _out_body, grid=(GRID,),
        in_specs=[_spec_part, _spec_deg, _spec_b, _spec_w, _spec_b],
        out_specs=_spec_rows,
        out_shape=jax.ShapeDtypeStruct((N, D), _f32),
    )(sp, degp, b2, wl, bl)


def kernel(x, edge_index, W1, b1, W2, b2, Wl, bl):
    src = edge_index[0]
    dst = edge_index[1]
    zeros128 = jnp.zeros((N, D), _f32)
    ones128 = jnp.ones((N, D), _f32)
    onesC = jnp.ones((C, D), _f32)
    b1r = b1.reshape(1, D)
    b2r = b2.reshape(1, D)
    blr = bl.reshape(1, D)

    degp = _sc_degrees(dst, ones128, zeros128, onesC)
    u1 = _tc_mm(x, W1)
    g1 = _tc_scale(u1, degp)
    s1 = _sc_aggregate(g1, src, dst, zeros128)
    g2 = _tc_comb2(s1, degp, b1r, W2)
    s2 = _sc_aggregate(g2, src, dst, zeros128)
    return _tc_out(s2, degp, b2r, Wl, blr)


# speedup vs baseline: 21.7463x; 1.7786x over previous
import functools

import jax
import jax.numpy as jnp
from jax import lax
from jax.experimental import pallas as pl
from jax.experimental.pallas import tpu as pltpu
from jax.experimental.pallas import tpu_sc as plsc

N = 10000
E = 320000
D = 128

NC = 2
NS = 16
NW = NC * NS
EPW = E // NW
C = 80
NCHUNK = EPW // C
RPT = 624
REM_BASE = NS * RPT
REM = N - REM_BASE

_mesh = plsc.VectorSubcoreMesh(
    core_axis_name="c", subcore_axis_name="s", num_cores=NC, num_subcores=NS
)


def _tile_rows_copy(s, fn):
    fn(s * RPT, RPT)

    @pl.when(s == NS - 1)
    def _():
        fn(REM_BASE, REM)


def _deg_body(dst_hbm, ones_hbm, zeros_hbm, onesC_hbm, out_hbm,
              idx_d0, idx_d1, ones_v, acc, semi0, semi1, sem):
    c = lax.axis_index("c")
    s = lax.axis_index("s")
    wid = s * NC + c

    @pl.when(c == 0)
    def _():
        _tile_rows_copy(s, lambda rb, nr: pltpu.sync_copy(
            ones_hbm.at[pl.ds(rb, nr)], acc.at[pl.ds(rb, nr)]))

    @pl.when(c != 0)
    def _():
        _tile_rows_copy(s, lambda rb, nr: pltpu.sync_copy(
            zeros_hbm.at[pl.ds(rb, nr)], acc.at[pl.ds(rb, nr)]))

    pltpu.sync_copy(onesC_hbm, ones_v)
    plsc.subcore_barrier()
    ebase = wid * EPW
    idx_d = (idx_d0, idx_d1)
    semi = (semi0, semi1)

    def idx_start(chunk, b):
        pltpu.async_copy(dst_hbm.at[pl.ds(ebase + chunk * C, C)], idx_d[b], semi[b])

    def idx_wait(chunk, b):
        pltpu.make_async_copy(
            dst_hbm.at[pl.ds(ebase + chunk * C, C)], idx_d[b], semi[b]).wait()

    idx_start(0, 0)
    idx_start(1, 1)

    @pl.loop(0, NCHUNK - 1, step=2)
    def _(k):
        for b in range(2):
            chunk = k + b
            idx_wait(chunk, b)
            pltpu.sync_copy(ones_v, acc.at[idx_d[b]], add=True)
            idx_start(chunk + 2, b)

    idx_wait(NCHUNK - 1, 0)
    pltpu.sync_copy(ones_v, acc.at[idx_d0], add=True)
    idx_wait(NCHUNK, 1)

    plsc.subcore_barrier()
    _tile_rows_copy(s, lambda rb, nr: pltpu.async_copy(
        acc.at[pl.ds(rb, nr)], out_hbm.at[c, pl.ds(rb, nr)], sem).wait())


@jax.jit
def _sc_degrees(dst, ones128, zeros128, onesC):
    kern = pl.kernel(
        _deg_body,
        out_type=jax.ShapeDtypeStruct((NC, N, D), jnp.float32),
        mesh=_mesh,
        scratch_types=[
            pltpu.VMEM((C,), jnp.int32),
            pltpu.VMEM((C,), jnp.int32),
            pltpu.VMEM((C, D), jnp.float32),
            pltpu.VMEM_SHARED((N, D), jnp.float32),
            pltpu.SemaphoreType.DMA,
            pltpu.SemaphoreType.DMA,
            pltpu.SemaphoreType.DMA,
        ],
    )
    return kern(dst, ones128, zeros128, onesC)


def _agg_body(g_hbm, src_hbm, dst_hbm, zeros_hbm, out_hbm,
              idx_s0, idx_s1, idx_d0, idx_d1, rows0, rows1, acc,
              semis0, semis1, semid0, semid1, semg0, semg1, sem):
    c = lax.axis_index("c")
    s = lax.axis_index("s")
    wid = s * NC + c

    @pl.when(c == 0)
    def _():
        _tile_rows_copy(s, lambda rb, nr: pltpu.sync_copy(
            g_hbm.at[pl.ds(rb, nr)], acc.at[pl.ds(rb, nr)]))

    @pl.when(c != 0)
    def _():
        _tile_rows_copy(s, lambda rb, nr: pltpu.sync_copy(
            zeros_hbm.at[pl.ds(rb, nr)], acc.at[pl.ds(rb, nr)]))

    plsc.subcore_barrier()
    ebase = wid * EPW
    idx_s = (idx_s0, idx_s1)
    idx_d = (idx_d0, idx_d1)
    rows = (rows0, rows1)
    semis = (semis0, semis1)
    semid = (semid0, semid1)
    semg = (semg0, semg1)

    def idx_start(chunk, b):
        base = ebase + chunk * C
        pltpu.async_copy(src_hbm.at[pl.ds(base, C)], idx_s[b], semis[b])
        pltpu.async_copy(dst_hbm.at[pl.ds(base, C)], idx_d[b], semid[b])

    def idx_wait(chunk, b):
        base = ebase + chunk * C
        pltpu.make_async_copy(src_hbm.at[pl.ds(base, C)], idx_s[b], semis[b]).wait()
        pltpu.make_async_copy(dst_hbm.at[pl.ds(base, C)], idx_d[b], semid[b]).wait()

    def g_start(b):
        pltpu.async_copy(g_hbm.at[idx_s[b]], rows[b], semg[b])

    def g_wait(b):
        pltpu.make_async_copy(g_hbm.at[idx_s[b]], rows[b], semg[b]).wait()

    idx_start(0, 0)
    idx_start(1, 1)
    idx_wait(0, 0)
    g_start(0)

    @pl.loop(0, NCHUNK - 1, step=2)
    def _(k):
        for b in range(2):
            chunk = k + b
            g_wait(b)
            idx_wait(chunk + 1, 1 - b)
            g_start(1 - b)
            pltpu.sync_copy(rows[b], acc.at[idx_d[b]], add=True)
            idx_start(chunk + 2, b)

    g_wait(0)
    pltpu.sync_copy(rows0, acc.at[idx_d0], add=True)
    idx_wait(NCHUNK, 1)

    plsc.subcore_barrier()
    _tile_rows_copy(s, lambda rb, nr: pltpu.async_copy(
        acc.at[pl.ds(rb, nr)], out_hbm.at[c, pl.ds(rb, nr)], sem).wait())


@jax.jit
def _sc_aggregate(g, src, dst, zeros128):
    kern = pl.kernel(
        _agg_body,
        out_type=jax.ShapeDtypeStruct((NC, N, D), jnp.float32),
        mesh=_mesh,
        scratch_types=[
            pltpu.VMEM((C,), jnp.int32),
            pltpu.VMEM((C,), jnp.int32),
            pltpu.VMEM((C,), jnp.int32),
            pltpu.VMEM((C,), jnp.int32),
            pltpu.VMEM((C, D), jnp.float32),
            pltpu.VMEM((C, D), jnp.float32),
            pltpu.VMEM_SHARED((N, D), jnp.float32),
            pltpu.SemaphoreType.DMA,
            pltpu.SemaphoreType.DMA,
            pltpu.SemaphoreType.DMA,
            pltpu.SemaphoreType.DMA,
            pltpu.SemaphoreType.DMA,
            pltpu.SemaphoreType.DMA,
            pltpu.SemaphoreType.DMA,
        ],
    )
    return kern(g, src, dst, zeros128)


BM = 1000
GRID = N // BM


def _mm_body(x_ref, w_ref, o_ref):
    o_ref[...] = jnp.dot(x_ref[...], w_ref[...],
                         preferred_element_type=jnp.float32)


def _scale_body(u_ref, degp_ref, o_ref):
    deg = degp_ref[0, :, 0:1] + degp_ref[1, :, 0:1]
    o_ref[...] = u_ref[...] * lax.rsqrt(deg)


def _comb2_body(sp_ref, degp_ref, b_ref, w_ref, o_ref):
    deg = degp_ref[0, :, 0:1] + degp_ref[1, :, 0:1]
    dis = lax.rsqrt(deg)
    h = jnp.maximum((sp_ref[0] + sp_ref[1]) * dis + b_ref[...], 0.0)
    o_ref[...] = jnp.dot(h, w_ref[...],
                         preferred_element_type=jnp.float32) * dis


def _out_body(sp_ref, degp_ref, b_ref, w_ref, bl_ref, o_ref):
    deg = degp_ref[0, :, 0:1] + degp_ref[1, :, 0:1]
    dis = lax.rsqrt(deg)
    h = (sp_ref[0] + sp_ref[1]) * dis + b_ref[...]
    o_ref[...] = jnp.dot(h, w_ref[...],
                         preferred_element_type=jnp.float32) + bl_ref[...]


_spec_rows = pl.BlockSpec((BM, D), lambda i: (i, 0))
_spec_part = pl.BlockSpec((NC, BM, D), lambda i: (0, i, 0))
_spec_deg = pl.BlockSpec((NC, BM, D), lambda i: (0, i, 0))
_spec_w = pl.BlockSpec((D, D), lambda i: (0, 0))
_spec_b = pl.BlockSpec((1, D), lambda i: (0, 0))
_f32 = jnp.float32


@jax.jit
def _tc_mm(x, w):
    return pl.pallas_call(
        _mm_body, grid=(GRID,),
        in_specs=[_spec_rows, _spec_w], out_specs=_spec_rows,
        out_shape=jax.ShapeDtypeStruct((N, D), _f32),
    )(x, w)


@jax.jit
def _tc_scale(u, degp):
    return pl.pallas_call(
        _scale_body, grid=(GRID,),
        in_specs=[_spec_rows, _spec_deg], out_specs=_spec_rows,
        out_shape=jax.ShapeDtypeStruct((N, D), _f32),
    )(u, degp)


@jax.jit
def _tc_comb2(sp, degp, b1, w2):
    return pl.pallas_call(
        _comb2_body, grid=(GRID,),
        in_specs=[_spec_part, _spec_deg, _spec_b, _spec_w],
        out_specs=_spec_rows,
        out_shape=jax.ShapeDtypeStruct((N, D), _f32),
    )(sp, degp, b1, w2)


@jax.jit
def _tc_out(sp, degp, b2, wl, bl):
    return pl.pallas_call(
        _out_body, grid=(GRID,),
        in_specs=[_spec_part, _spec_deg, _spec_b, _spec_w, _spec_b],
        out_specs=_spec_rows,
        out_shape=jax.ShapeDtypeStruct((N, D), _f32),
    )(sp, degp, b2, wl, bl)


def kernel(x, edge_index, W1, b1, W2, b2, Wl, bl):
    pad = jnp.zeros((2 * C,), edge_index.dtype)
    src = jnp.concatenate([edge_index[0], pad])
    dst = jnp.concatenate([edge_index[1], pad])
    zeros128 = jnp.zeros((N, D), _f32)
    ones128 = jnp.ones((N, D), _f32)
    onesC = jnp.ones((C, D), _f32)
    b1r = b1.reshape(1, D)
    b2r = b2.reshape(1, D)
    blr = bl.reshape(1, D)

    degp = _sc_degrees(dst, ones128, zeros128, onesC)
    u1 = _tc_mm(x, W1)
    g1 = _tc_scale(u1, degp)
    s1 = _sc_aggregate(g1, src, dst, zeros128)
    g2 = _tc_comb2(s1, degp, b1r, W2)
    s2 = _sc_aggregate(g2, src, dst, zeros128)
    return _tc_out(s2, degp, b2r, Wl, blr)
